# baseline (speedup 1.0000x reference)
import functools

import jax
import jax.numpy as jnp
from jax import lax
from jax.experimental import pallas as pl
from jax.experimental.pallas import tpu as pltpu
from jax.experimental.pallas import tpu_sc as plsc

N = 884
FD = 128
E = 56576
NP = 888
KB = 7
SLOT = KB * NP * 128
CH = 2 * SLOT // 16
ZB = 4144
NZC = CH // ZB
NW = 32
EPW = 5376
NCH = EPW // 128
TOT = NW * EPW
HALF = 16 * EPW
DUMP = N * 128


def _sc_body(sidx_hbm, out_hbm, s_v, ones_v, zbuf, a_sh, zsem, ssem):
    c = lax.axis_index("c")
    s = lax.axis_index("s")
    w = c * 16 + s

    def _zb(i, carry):
        zbuf[pl.ds(i * 16, 16)] = jnp.zeros((16,), jnp.float32)
        return carry

    lax.fori_loop(0, ZB // 16, _zb, 0)
    for i in range(8):
        ones_v[pl.ds(i * 16, 16)] = jnp.ones((16,), jnp.float32)

    base = pl.multiple_of(s * CH, 8)
    zcopies = [
        pltpu.async_copy(zbuf, a_sh.at[pl.ds(base + k * ZB, ZB)], zsem)
        for k in range(NZC)
    ]
    pltpu.sync_copy(sidx_hbm.at[w], s_v)
    for cp in zcopies:
        cp.wait()
    plsc.subcore_barrier()

    def _fire(j, carry):
        pltpu.async_copy(ones_v, a_sh.at[s_v.at[j]], ssem, add=True)
        return carry

    lax.fori_loop(0, NCH, _fire, 0)

    def _drain(j, carry):
        pltpu.make_async_copy(ones_v, a_sh.at[s_v.at[j]], ssem).wait()
        return carry

    lax.fori_loop(0, NCH, _drain, 0)
    plsc.subcore_barrier()

    pltpu.sync_copy(a_sh.at[pl.ds(base, CH)],
                    out_hbm.at[pl.ds(c * 2 * SLOT + base, CH)])


@functools.cache
def _sc_build_counts():
    return pl.kernel(
        _sc_body,
        out_type=jax.ShapeDtypeStruct((4 * SLOT,), jnp.float32),
        mesh=plsc.VectorSubcoreMesh(core_axis_name="c", subcore_axis_name="s"),
        scratch_types=[
            pltpu.VMEM((NCH, 128), jnp.int32),
            pltpu.VMEM((128,), jnp.float32),
            pltpu.VMEM((ZB,), jnp.float32),
            pltpu.VMEM_SHARED((2 * SLOT,), jnp.float32),
            pltpu.SemaphoreType.DMA,
            pltpu.SemaphoreType.DMA,
        ],
    )


def _tc_body(x_ref, scr_ref, dk0_ref, dk1_ref, dk2_ref,
             w1_refs, b1_refs, w2_refs, b2_refs,
             f1w_ref, f1b_ref, f2w_ref, f2b_ref, cw_ref, cb_ref, out_ref):
    x = x_ref[...]
    rowmask = jnp.where(
        lax.broadcasted_iota(jnp.int32, (NP, 1), 0) < N, 1.0, 0.0)

    ys = []
    msums = []
    for v in range(3):
        if v == 0:
            C = scr_ref[0]
            Dk = dk0_ref[...]
        elif v == 1:
            C = scr_ref[1] + scr_ref[2]
            Dk = dk1_ref[...]
        else:
            C = scr_ref[3]
            Dk = dk2_ref[...]
        B = C * Dk
        deg = jnp.sum(jnp.sum(B, axis=2, keepdims=True), axis=0) + 1.0
        dinv = jnp.where(deg > 0, lax.rsqrt(deg), 0.0)
        h = x
        for W_ref, b_ref in ((w1_refs[v], b1_refs[v]), (w2_refs[v], b2_refs[v])):
            hw = jnp.dot(h, W_ref[...], preferred_element_type=jnp.float32)
            z = dinv * hw
            zp = jnp.concatenate(
                [z, jnp.zeros((KB * 128 - NP, FD), jnp.float32)], axis=0)
            agg = z
            for k in range(KB):
                agg = agg + jnp.dot(B[k], zp[128 * k:128 * (k + 1), :],
                                    preferred_element_type=jnp.float32)
            h = jnp.maximum(dinv * agg + b_ref[...], 0.0) * rowmask
            ys.append(h)
            msums.append(jnp.sum(h))

    m = jnp.concatenate([t.reshape(1, 1) for t in msums], axis=1) / (N * FD)
    ca = jnp.maximum(
        jnp.dot(m, f1w_ref[...], preferred_element_type=jnp.float32)
        + f1b_ref[...], 0.0)
    ca = jax.nn.sigmoid(
        jnp.dot(ca, f2w_ref[...], preferred_element_type=jnp.float32)
        + f2b_ref[...])

    acc = jnp.full((NP, FD), cb_ref[0, 0], jnp.float32)
    for j in range(6):
        acc = acc + cw_ref[0, j] * jnp.maximum(ca[0, j] * ys[j], 0.0)
    out_ref[...] = acc[:N, :]


def _blocked(D):
    dt = jnp.pad(D.T, ((0, NP - N), (0, KB * 128 - N)))
    return jnp.transpose(dt.reshape(NP, KB, 128), (1, 0, 2))


def kernel(x_d, di_gua, di_cos, di_sem, W_t1, b_t1, W_t2, b_t2, W_s1, b_s1,
           W_s2, b_s2, W_g1, b_g1, W_g2, b_g2, fc1_W, fc1_b, fc2_W, fc2_b,
           cnn_W, cnn_b, di_gua_edges, di_cos_edges, di_sem_edges):
    f32 = jnp.float32

    def _addr(edges, slot):
        s, dd = edges[0], edges[1]
        return (s // 128) * (NP * 128) + dd * 128 + (s % 128) + slot

    q = jnp.arange(E, dtype=jnp.int32)
    slot1 = jnp.where(q < HALF - E, SLOT, 0).astype(jnp.int32)
    npad = TOT - 3 * E
    s_pad = jnp.concatenate([
        _addr(di_gua_edges, 0),
        _addr(di_cos_edges, slot1),
        _addr(di_sem_edges, SLOT),
        jnp.full((npad,), DUMP, jnp.int32),
    ])
    sidx3 = s_pad.reshape(NW, NCH, 128)

    dk0, dk1, dk2 = _blocked(di_gua), _blocked(di_cos), _blocked(di_sem)
    xp = jnp.pad(x_d, ((0, NP - N), (0, 0)))

    sc_out = _sc_build_counts()(sidx3)
    scr = sc_out.reshape(4, KB, NP, 128)

    out = pl.pallas_call(
        _tc_body,
        out_shape=jax.ShapeDtypeStruct((N, FD), f32),
    )(
        xp, scr, dk0, dk1, dk2,
        [W_t1, W_s1, W_g1], [b_t1.reshape(1, FD), b_s1.reshape(1, FD),
                             b_g1.reshape(1, FD)],
        [W_t2, W_s2, W_g2], [b_t2.reshape(1, FD), b_s2.reshape(1, FD),
                             b_g2.reshape(1, FD)],
        fc1_W, fc1_b.reshape(1, 30), fc2_W, fc2_b.reshape(1, 6),
        cnn_W.reshape(1, 6), cnn_b.reshape(1, 1),
    )
    return out


# speedup vs baseline: 1.0748x; 1.0748x over previous
import functools

import jax
import jax.numpy as jnp
from jax import lax
from jax.experimental import pallas as pl
from jax.experimental.pallas import tpu as pltpu
from jax.experimental.pallas import tpu_sc as plsc

N = 884
FD = 128
E = 56576
NP = 888
KB = 7
SLOT = KB * NP * 128
CH = 2 * SLOT // 16
ZB = 4144
NZC = CH // ZB
NW = 32
EPW = 5376
NCH = EPW // 128
TOT = NW * EPW
HALF = 16 * EPW
DUMP = N * 128


def _sc_body(sidx_hbm, out_hbm, s_v, ones_v, zbuf, a_sh, zsem, ssem):
    c = lax.axis_index("c")
    s = lax.axis_index("s")
    w = c * 16 + s

    def _zb(i, carry):
        zbuf[pl.ds(i * 16, 16)] = jnp.zeros((16,), jnp.float32)
        return carry

    lax.fori_loop(0, ZB // 16, _zb, 0)
    for i in range(8):
        ones_v[pl.ds(i * 16, 16)] = jnp.ones((16,), jnp.float32)

    base = pl.multiple_of(s * CH, 8)
    zcopies = [
        pltpu.async_copy(zbuf, a_sh.at[pl.ds(base + k * ZB, ZB)], zsem)
        for k in range(NZC)
    ]
    pltpu.sync_copy(sidx_hbm.at[w], s_v)
    for cp in zcopies:
        cp.wait()
    plsc.subcore_barrier()

    def _fire(j, carry):
        pltpu.async_copy(ones_v, a_sh.at[s_v.at[j]], ssem, add=True)
        return carry

    lax.fori_loop(0, NCH, _fire, 0)

    def _drain(j, carry):
        pltpu.make_async_copy(ones_v, a_sh.at[s_v.at[j]], ssem).wait()
        return carry

    lax.fori_loop(0, NCH, _drain, 0)
    plsc.subcore_barrier()

    pltpu.sync_copy(a_sh.at[pl.ds(base, CH)],
                    out_hbm.at[pl.ds(c * 2 * SLOT + base, CH)])


@functools.cache
def _sc_build_counts():
    return pl.kernel(
        _sc_body,
        out_type=jax.ShapeDtypeStruct((4 * SLOT,), jnp.float32),
        mesh=plsc.VectorSubcoreMesh(core_axis_name="c", subcore_axis_name="s"),
        scratch_types=[
            pltpu.VMEM((NCH, 128), jnp.int32),
            pltpu.VMEM((128,), jnp.float32),
            pltpu.VMEM((ZB,), jnp.float32),
            pltpu.VMEM_SHARED((2 * SLOT,), jnp.float32),
            pltpu.SemaphoreType.DMA,
            pltpu.SemaphoreType.DMA,
        ],
    )


def _tc_body(x_ref, scr_ref, dk0_ref, dk1_ref, dk2_ref,
             w1_refs, b1_refs, w2_refs, b2_refs,
             f1w_ref, f1b_ref, f2w_ref, f2b_ref, cw_ref, cb_ref, out_ref):
    x = x_ref[...]
    rowmask = jnp.where(
        lax.broadcasted_iota(jnp.int32, (NP, 1), 0) < N, 1.0, 0.0)

    ys = []
    msums = []
    for v in range(3):
        if v == 0:
            C = scr_ref[0]
            Dk = dk0_ref[...].astype(jnp.float32)
        elif v == 1:
            C = scr_ref[1] + scr_ref[2]
            Dk = dk1_ref[...].astype(jnp.float32)
        else:
            C = scr_ref[3]
            Dk = dk2_ref[...].astype(jnp.float32)
        B = C * Dk
        deg = jnp.sum(jnp.sum(B, axis=2, keepdims=True), axis=0) + 1.0
        dinv = jnp.where(deg > 0, lax.rsqrt(deg), 0.0)
        h = x
        for W_ref, b_ref in ((w1_refs[v], b1_refs[v]), (w2_refs[v], b2_refs[v])):
            hw = jnp.dot(h, W_ref[...], preferred_element_type=jnp.float32)
            z = dinv * hw
            zp = jnp.concatenate(
                [z, jnp.zeros((KB * 128 - NP, FD), jnp.float32)], axis=0)
            agg = z
            for k in range(KB):
                agg = agg + jnp.dot(B[k], zp[128 * k:128 * (k + 1), :],
                                    preferred_element_type=jnp.float32)
            h = jnp.maximum(dinv * agg + b_ref[...], 0.0) * rowmask
            ys.append(h)
            msums.append(jnp.sum(h))

    m = jnp.concatenate([t.reshape(1, 1) for t in msums], axis=1) / (N * FD)
    ca = jnp.maximum(
        jnp.dot(m, f1w_ref[...], preferred_element_type=jnp.float32)
        + f1b_ref[...], 0.0)
    ca = jax.nn.sigmoid(
        jnp.dot(ca, f2w_ref[...], preferred_element_type=jnp.float32)
        + f2b_ref[...])

    acc = jnp.full((NP, FD), cb_ref[0, 0], jnp.float32)
    for j in range(6):
        acc = acc + cw_ref[0, j] * jnp.maximum(ca[0, j] * ys[j], 0.0)
    out_ref[...] = acc[:N, :]


def _blocked(D):
    dt = jnp.pad(D.T, ((0, NP - N), (0, KB * 128 - N)))
    return jnp.transpose(dt.reshape(NP, KB, 128), (1, 0, 2)).astype(jnp.bfloat16)


def kernel(x_d, di_gua, di_cos, di_sem, W_t1, b_t1, W_t2, b_t2, W_s1, b_s1,
           W_s2, b_s2, W_g1, b_g1, W_g2, b_g2, fc1_W, fc1_b, fc2_W, fc2_b,
           cnn_W, cnn_b, di_gua_edges, di_cos_edges, di_sem_edges):
    f32 = jnp.float32

    def _addr(edges, slot):
        s, dd = edges[0], edges[1]
        return (s // 128) * (NP * 128) + dd * 128 + (s % 128) + slot

    q = jnp.arange(E, dtype=jnp.int32)
    slot1 = jnp.where(q < HALF - E, SLOT, 0).astype(jnp.int32)
    npad = TOT - 3 * E
    s_pad = jnp.concatenate([
        _addr(di_gua_edges, 0),
        _addr(di_cos_edges, slot1),
        _addr(di_sem_edges, SLOT),
        jnp.full((npad,), DUMP, jnp.int32),
    ])
    sidx3 = s_pad.reshape(NW, NCH, 128)

    dk0, dk1, dk2 = _blocked(di_gua), _blocked(di_cos), _blocked(di_sem)
    xp = jnp.pad(x_d, ((0, NP - N), (0, 0)))

    sc_out = _sc_build_counts()(sidx3)
    scr = sc_out.reshape(4, KB, NP, 128)

    out = pl.pallas_call(
        _tc_body,
        out_shape=jax.ShapeDtypeStruct((N, FD), f32),
    )(
        xp, scr, dk0, dk1, dk2,
        [W_t1, W_s1, W_g1], [b_t1.reshape(1, FD), b_s1.reshape(1, FD),
                             b_g1.reshape(1, FD)],
        [W_t2, W_s2, W_g2], [b_t2.reshape(1, FD), b_s2.reshape(1, FD),
                             b_g2.reshape(1, FD)],
        fc1_W, fc1_b.reshape(1, 30), fc2_W, fc2_b.reshape(1, 6),
        cnn_W.reshape(1, 6), cnn_b.reshape(1, 1),
    )
    return out
